# baseline (speedup 1.0000x reference)
import functools

import jax
import jax.numpy as jnp
from jax import lax
from jax.experimental import pallas as pl
from jax.experimental.pallas import tpu as pltpu
from jax.experimental.pallas import tpu_sc as plsc

D_MODEL = 64
ROWS_PER_DMA = 128
DMAS_PER_CHUNK = 8
CHUNK = ROWS_PER_DMA * DMAS_PER_CHUNK


@functools.lru_cache(maxsize=None)
def _make_gather(vocab: int, batch: int):
    info = plsc.get_sparse_core_info()
    num_workers = info.num_cores * info.num_subcores
    assert batch % (num_workers * CHUNK) == 0
    rows_per_worker = batch // num_workers
    n_outer = rows_per_worker // CHUNK

    mesh = plsc.VectorSubcoreMesh(core_axis_name="c", subcore_axis_name="s")

    @functools.partial(
        pl.kernel,
        mesh=mesh,
        compiler_params=pltpu.CompilerParams(use_tc_tiling_on_sc=False),
        out_type=jax.ShapeDtypeStruct((batch, D_MODEL), jnp.float32),
        scratch_types=[
            pltpu.VMEM((CHUNK,), jnp.int32),
            pltpu.VMEM((CHUNK, D_MODEL), jnp.float32),
            pltpu.SemaphoreType.DMA,
        ],
    )
    def gather_kernel(idx_hbm, table_hbm, out_hbm, idx_v, rows_v, sem):
        wid = lax.axis_index("s") * info.num_cores + lax.axis_index("c")
        base = wid * rows_per_worker

        def body(c, carry):
            off = base + c * CHUNK
            pltpu.sync_copy(idx_hbm.at[pl.ds(off, CHUNK)], idx_v)
            copies = []
            for j in range(DMAS_PER_CHUNK):
                copies.append(
                    pltpu.async_copy(
                        table_hbm.at[idx_v.at[pl.ds(j * ROWS_PER_DMA, ROWS_PER_DMA)]],
                        rows_v.at[pl.ds(j * ROWS_PER_DMA, ROWS_PER_DMA)],
                        sem,
                    )
                )
            for cp in copies:
                cp.wait()
            pltpu.sync_copy(rows_v, out_hbm.at[pl.ds(off, CHUNK)])
            return carry

        lax.fori_loop(0, n_outer, body, 0)

    return gather_kernel


def kernel(x, table):
    b0, b1 = x.shape
    batch = b0 * b1
    flat_idx = x.reshape(batch).astype(jnp.int32)
    out = _make_gather(table.shape[0], batch)(flat_idx, table)
    return out.reshape(b0, b1, D_MODEL)


# speedup vs baseline: 1.0169x; 1.0169x over previous
import functools

import jax
import jax.numpy as jnp
from jax import lax
from jax.experimental import pallas as pl
from jax.experimental.pallas import tpu as pltpu
from jax.experimental.pallas import tpu_sc as plsc

D_MODEL = 64
ROWS_PER_DMA = 128
DMAS_PER_CHUNK = 4
CHUNK = ROWS_PER_DMA * DMAS_PER_CHUNK


@functools.lru_cache(maxsize=None)
def _make_gather(vocab: int, batch: int):
    info = plsc.get_sparse_core_info()
    num_workers = info.num_cores * info.num_subcores
    assert batch % (num_workers * 2 * CHUNK) == 0
    rows_per_worker = batch // num_workers
    n_chunks = rows_per_worker // CHUNK

    mesh = plsc.VectorSubcoreMesh(core_axis_name="c", subcore_axis_name="s")

    @functools.partial(
        pl.kernel,
        mesh=mesh,
        compiler_params=pltpu.CompilerParams(use_tc_tiling_on_sc=False),
        out_type=jax.ShapeDtypeStruct((batch, D_MODEL), jnp.float32),
        scratch_types=[
            pltpu.VMEM((rows_per_worker,), jnp.int32),
            pltpu.VMEM((CHUNK, D_MODEL), jnp.float32),
            pltpu.VMEM((CHUNK, D_MODEL), jnp.float32),
            pltpu.SemaphoreType.DMA,
            pltpu.SemaphoreType.DMA,
            pltpu.SemaphoreType.DMA,
        ],
    )
    def gather_kernel(idx_hbm, table_hbm, out_hbm, idx_v, rows0, rows1,
                      gsem, wsem0, wsem1):
        wid = lax.axis_index("s") * info.num_cores + lax.axis_index("c")
        base = wid * rows_per_worker
        pltpu.sync_copy(idx_hbm.at[pl.ds(base, rows_per_worker)], idx_v)

        rows_bufs = (rows0, rows1)
        wsems = (wsem0, wsem1)

        def do_chunk(c, b, wait_prev):
            rows_b = rows_bufs[b]
            wsem_b = wsems[b]
            off = base + c * CHUNK
            if wait_prev:
                pltpu.make_async_copy(
                    rows_b, out_hbm.at[pl.ds(off - 2 * CHUNK, CHUNK)], wsem_b
                ).wait()
            copies = []
            for j in range(DMAS_PER_CHUNK):
                copies.append(
                    pltpu.async_copy(
                        table_hbm.at[
                            idx_v.at[pl.ds(c * CHUNK + j * ROWS_PER_DMA,
                                           ROWS_PER_DMA)]
                        ],
                        rows_b.at[pl.ds(j * ROWS_PER_DMA, ROWS_PER_DMA)],
                        gsem,
                    )
                )
            for cp in copies:
                cp.wait()
            pltpu.async_copy(rows_b, out_hbm.at[pl.ds(off, CHUNK)], wsem_b)

        do_chunk(0, 0, False)
        do_chunk(1, 1, False)

        def body(i, carry):
            do_chunk(2 * i, 0, True)
            do_chunk(2 * i + 1, 1, True)
            return carry

        lax.fori_loop(1, n_chunks // 2, body, 0)

        pltpu.make_async_copy(
            rows0, out_hbm.at[pl.ds(base + (n_chunks - 2) * CHUNK, CHUNK)], wsem0
        ).wait()
        pltpu.make_async_copy(
            rows1, out_hbm.at[pl.ds(base + (n_chunks - 1) * CHUNK, CHUNK)], wsem1
        ).wait()

    return gather_kernel


def kernel(x, table):
    b0, b1 = x.shape
    batch = b0 * b1
    flat_idx = x.reshape(batch).astype(jnp.int32)
    out = _make_gather(table.shape[0], batch)(flat_idx, table)
    return out.reshape(b0, b1, D_MODEL)


# speedup vs baseline: 1.0473x; 1.0298x over previous
import functools

import jax
import jax.numpy as jnp
from jax import lax
from jax.experimental import pallas as pl
from jax.experimental.pallas import tpu as pltpu
from jax.experimental.pallas import tpu_sc as plsc

D_MODEL = 64
NBUF = 4


@functools.lru_cache(maxsize=None)
def _make_gather(vocab: int, n_j: int, n_i: int):
    info = plsc.get_sparse_core_info()
    num_workers = info.num_cores * info.num_subcores
    rows = n_i // num_workers
    assert n_i % num_workers == 0 and rows <= 128 and n_j % NBUF == 0

    mesh = plsc.VectorSubcoreMesh(core_axis_name="c", subcore_axis_name="s")

    @functools.partial(
        pl.kernel,
        mesh=mesh,
        compiler_params=pltpu.CompilerParams(use_tc_tiling_on_sc=False),
        out_type=jax.ShapeDtypeStruct((n_j, n_i, D_MODEL), jnp.float32),
        scratch_types=[
            pltpu.VMEM((n_j, rows), jnp.int32),
        ]
        + [pltpu.VMEM((rows, D_MODEL), jnp.float32)] * NBUF
        + [pltpu.SemaphoreType.DMA] * (2 * NBUF),
    )
    def gather_kernel(idx_hbm, table_hbm, out_hbm, idx_v,
                      g0, g1, g2, g3, gs0, gs1, gs2, gs3, ws0, ws1, ws2, ws3):
        wid = lax.axis_index("s") * info.num_cores + lax.axis_index("c")
        base = wid * rows
        grow = (g0, g1, g2, g3)
        gsem = (gs0, gs1, gs2, gs3)
        wsem = (ws0, ws1, ws2, ws3)

        pltpu.sync_copy(idx_hbm.at[:, pl.ds(base, rows)], idx_v)

        def fire_gather(j, b):
            pltpu.async_copy(table_hbm.at[idx_v.at[j]], grow[b], gsem[b])

        def wait_gather(b):
            pltpu.make_async_copy(
                out_hbm.at[0, pl.ds(base, rows)], grow[b], gsem[b]
            ).wait()

        def fire_wb(j, b):
            pltpu.async_copy(grow[b], out_hbm.at[j, pl.ds(base, rows)], wsem[b])

        def wait_wb(j, b):
            pltpu.make_async_copy(
                grow[b], out_hbm.at[j, pl.ds(base, rows)], wsem[b]
            ).wait()

        for j in range(NBUF - 1):
            fire_gather(j, j)

        wait_gather(0)
        fire_wb(0, 0)
        fire_gather(NBUF - 1, NBUF - 1)

        def body(k, carry):
            for m in range(NBUF):
                j = NBUF * k + 1 + m
                b = (1 + m) % NBUF
                wait_gather(b)
                fire_wb(j, b)
                wait_wb(j - 1, (b - 1) % NBUF)
                fire_gather(j + NBUF - 1, (b + NBUF - 1) % NBUF)
            return carry

        lax.fori_loop(0, (n_j - NBUF) // NBUF, body, 0)

        for m in range(NBUF - 1):
            j = n_j - (NBUF - 1) + m
            b = j % NBUF
            wait_gather(b)
            fire_wb(j, b)
        for m in range(NBUF):
            j = n_j - NBUF + m
            wait_wb(j, j % NBUF)

    return gather_kernel


def kernel(x, table):
    n_i, n_j = x.shape
    xt = jnp.transpose(x, (1, 0)).astype(jnp.int32)
    out = _make_gather(table.shape[0], n_j, n_i)(xt, table)
    return jnp.transpose(out, (1, 0, 2))


# speedup vs baseline: 1.3727x; 1.3107x over previous
import functools

import jax
import jax.numpy as jnp
from jax import lax
from jax.experimental import pallas as pl
from jax.experimental.pallas import tpu as pltpu
from jax.experimental.pallas import tpu_sc as plsc

D_MODEL = 64
NBUF = 4


@functools.lru_cache(maxsize=None)
def _make_gather(vocab: int, n_j: int, n_i: int):
    info = plsc.get_sparse_core_info()
    num_workers = info.num_cores * info.num_subcores
    rows = n_i // num_workers
    assert n_i % num_workers == 0 and rows <= 128 and n_j % NBUF == 0

    mesh = plsc.VectorSubcoreMesh(core_axis_name="c", subcore_axis_name="s")

    @functools.partial(
        pl.kernel,
        mesh=mesh,
        out_type=jax.ShapeDtypeStruct((n_i, n_j, D_MODEL), jnp.float32),
        scratch_types=[
            pltpu.VMEM((n_j, rows), jnp.int32),
        ]
        + [pltpu.VMEM((rows, D_MODEL), jnp.float32)] * NBUF
        + [pltpu.SemaphoreType.DMA] * (2 * NBUF),
    )
    def gather_kernel(idx_hbm, table_hbm, out_hbm, idx_v,
                      g0, g1, g2, g3, gs0, gs1, gs2, gs3, ws0, ws1, ws2, ws3):
        wid = lax.axis_index("s") * info.num_cores + lax.axis_index("c")
        base = wid * rows
        grow = (g0, g1, g2, g3)
        gsem = (gs0, gs1, gs2, gs3)
        wsem = (ws0, ws1, ws2, ws3)

        pltpu.sync_copy(idx_hbm.at[:, pl.ds(base, rows)], idx_v)

        def fire_gather(j, b):
            def grp_body(g, carry):
                vec = idx_v[j, pl.ds(g * 16, 16)]
                for l in range(16):
                    pltpu.async_copy(
                        table_hbm.at[pl.ds(vec[l], 1)],
                        grow[b].at[pl.ds(g * 16 + l, 1)],
                        gsem[b],
                    )
                return carry
            lax.fori_loop(0, rows // 16, grp_body, 0)

        def wait_gather(b):
            pltpu.make_async_copy(
                out_hbm.at[pl.ds(base, rows), 0], grow[b], gsem[b]
            ).wait()

        def fire_wb(j, b):
            pltpu.async_copy(grow[b], out_hbm.at[pl.ds(base, rows), j], wsem[b])

        def wait_wb(j, b):
            pltpu.make_async_copy(
                grow[b], out_hbm.at[pl.ds(base, rows), j], wsem[b]
            ).wait()

        for j in range(NBUF - 1):
            fire_gather(j, j)

        wait_gather(0)
        fire_wb(0, 0)
        fire_gather(NBUF - 1, NBUF - 1)

        def body(k, carry):
            for m in range(NBUF):
                j = NBUF * k + 1 + m
                b = (1 + m) % NBUF
                wait_gather(b)
                fire_wb(j, b)
                wait_wb(j - 1, (b - 1) % NBUF)
                fire_gather(j + NBUF - 1, (b + NBUF - 1) % NBUF)
            return carry

        lax.fori_loop(0, (n_j - NBUF) // NBUF, body, 0)

        for m in range(NBUF - 1):
            j = n_j - (NBUF - 1) + m
            b = j % NBUF
            wait_gather(b)
            fire_wb(j, b)
        for m in range(NBUF):
            j = n_j - NBUF + m
            wait_wb(j, j % NBUF)

    return gather_kernel


def kernel(x, table):
    n_i, n_j = x.shape
    xt = jnp.transpose(x, (1, 0)).astype(jnp.int32)
    return _make_gather(table.shape[0], n_j, n_i)(xt, table)
